# baseline (speedup 1.0000x reference)
import functools

import jax
import jax.numpy as jnp
from jax import lax
from jax.experimental import pallas as pl
from jax.experimental.pallas import tpu as pltpu
from jax.experimental.pallas import tpu_sc as plsc

N_ROWS = 32768
N_COLS = 64
K = 8
VPAD = 65
OPAD = 9
NC = 2
NS = 16
L = 16
NW = NC * NS
RPW = N_ROWS // NW
GROUPS = RPW // L


def _sc_body(logits_hbm, idx_hbm, prob_hbm, vals_v, idx_v, val_v, prob_v):
    wid = lax.axis_index("s") * NC + lax.axis_index("c")
    base = wid * RPW
    pltpu.sync_copy(logits_hbm.at[pl.ds(base * VPAD, RPW * VPAD)], vals_v)

    lane = lax.iota(jnp.int32, L)
    cint = [jnp.full((L,), j, jnp.int32) for j in range(N_COLS)]

    sort8_net = [(0, 1), (2, 3), (4, 5), (6, 7),
                 (0, 2), (1, 3), (4, 6), (5, 7),
                 (1, 2), (5, 6),
                 (0, 4), (1, 5), (2, 6), (3, 7),
                 (2, 4), (3, 5),
                 (1, 2), (3, 4), (5, 6)]
    bitonic8_net = [(0, 4), (1, 5), (2, 6), (3, 7),
                    (0, 2), (1, 3), (4, 6), (5, 7),
                    (0, 1), (2, 3), (4, 5), (6, 7)]

    def apply_net(v, net):
        for i, j in net:
            lo = jnp.minimum(v[i], v[j])
            hi = jnp.maximum(v[i], v[j])
            v[i], v[j] = lo, hi
        return v

    @plsc.parallel_loop(0, GROUPS, unroll=2)
    def group(g):
        vbase = (g * L + lane) * VPAD
        obase = (g * L + lane) * OPAD

        def load_block(b):
            return [plsc.load_gather(vals_v, [vbase + cint[8 * b + u]])
                    for u in range(K)]

        run = apply_net(load_block(0), sort8_net)
        for b in range(1, N_COLS // K):
            blk = apply_net(load_block(b), sort8_net)
            c = [jnp.maximum(run[i], blk[K - 1 - i]) for i in range(K)]
            run = apply_net(c, bitonic8_net)
        t = run[0]
        m = run[K - 1]
        regs = run

        eq_budget = jnp.zeros((L,), jnp.int32)
        for r in regs:
            eq_budget = eq_budget + jnp.where(r == t, 1, 0)
        eq_seen = jnp.zeros((L,), jnp.int32)
        cnt = obase
        for j in range(N_COLS):
            x = plsc.load_gather(vals_v, [vbase + cint[j]])
            is_eq = x == t
            sel = jnp.logical_or(x > t,
                                 jnp.logical_and(is_eq, eq_seen < eq_budget))
            plsc.store_scatter(idx_v, [cnt], cint[j], mask=sel)
            plsc.store_scatter(val_v, [cnt], x, mask=sel)
            cnt = cnt + jnp.where(sel, 1, 0)
            eq_seen = eq_seen + jnp.where(is_eq, 1, 0)

        es = []
        denom = jnp.zeros((L,), jnp.float32)
        for p in range(K):
            vp = plsc.load_gather(val_v, [obase + cint[p]])
            e = jnp.exp(vp - m)
            es.append(e)
            denom = denom + e
        inv = 1.0 / denom
        for p in range(K):
            plsc.store_scatter(prob_v, [obase + cint[p]], es[p] * inv)

    pltpu.sync_copy(idx_v, idx_hbm.at[pl.ds(base * OPAD, RPW * OPAD)])
    pltpu.sync_copy(prob_v, prob_hbm.at[pl.ds(base * OPAD, RPW * OPAD)])


_sc_call = functools.partial(
    pl.kernel,
    out_type=(
        jax.ShapeDtypeStruct((N_ROWS * OPAD,), jnp.int32),
        jax.ShapeDtypeStruct((N_ROWS * OPAD,), jnp.float32),
    ),
    mesh=plsc.VectorSubcoreMesh(
        core_axis_name="c", subcore_axis_name="s",
        num_cores=NC, num_subcores=NS,
    ),
    compiler_params=pltpu.CompilerParams(needs_layout_passes=False),
    scratch_types=[
        pltpu.VMEM((RPW * VPAD,), jnp.float32),
        pltpu.VMEM((RPW * OPAD,), jnp.int32),
        pltpu.VMEM((RPW * OPAD,), jnp.float32),
        pltpu.VMEM((RPW * OPAD,), jnp.float32),
    ],
)(_sc_body)


def kernel(logits):
    padded = jnp.pad(logits, ((0, 0), (0, VPAD - N_COLS))).reshape(-1)
    idx_p, prob_p = _sc_call(padded)
    return (idx_p.reshape(N_ROWS, OPAD)[:, :K],
            prob_p.reshape(N_ROWS, OPAD)[:, :K])


# speedup vs baseline: 1.0725x; 1.0725x over previous
import functools

import jax
import jax.numpy as jnp
from jax import lax
from jax.experimental import pallas as pl
from jax.experimental.pallas import tpu as pltpu
from jax.experimental.pallas import tpu_sc as plsc

N_ROWS = 32768
N_COLS = 64
K = 8
VPAD = 65
OPAD = 9
NC = 2
NS = 16
L = 16
NW = NC * NS
RPW = N_ROWS // NW
GROUPS = RPW // L


def _sc_body(logits_hbm, idx_hbm, prob_hbm, vals_v, idx_v, val_v, prob_v):
    wid = lax.axis_index("s") * NC + lax.axis_index("c")
    base = wid * RPW
    pltpu.sync_copy(logits_hbm.at[pl.ds(base * VPAD, RPW * VPAD)], vals_v)

    lane = lax.iota(jnp.int32, L)
    cint = [jnp.full((L,), j, jnp.int32) for j in range(N_COLS)]

    sort8_net = [(0, 1), (2, 3), (4, 5), (6, 7),
                 (0, 2), (1, 3), (4, 6), (5, 7),
                 (1, 2), (5, 6),
                 (0, 4), (1, 5), (2, 6), (3, 7),
                 (2, 4), (3, 5),
                 (1, 2), (3, 4), (5, 6)]
    bitonic8_net = [(0, 4), (1, 5), (2, 6), (3, 7),
                    (0, 2), (1, 3), (4, 6), (5, 7),
                    (0, 1), (2, 3), (4, 5), (6, 7)]

    def apply_net(v, net):
        for i, j in net:
            lo = jnp.minimum(v[i], v[j])
            hi = jnp.maximum(v[i], v[j])
            v[i], v[j] = lo, hi
        return v

    @plsc.parallel_loop(0, GROUPS, unroll=1)
    def group(g):
        vbase = (g * L + lane) * VPAD
        obase = (g * L + lane) * OPAD

        def load_block(b):
            return [plsc.load_gather(vals_v, [vbase + cint[8 * b + u]])
                    for u in range(K)]

        run = apply_net(load_block(0), sort8_net)
        for b in range(1, N_COLS // K):
            blk = apply_net(load_block(b), sort8_net)
            c = [jnp.maximum(run[i], blk[K - 1 - i]) for i in range(K)]
            run = apply_net(c, bitonic8_net)
        t = run[0]
        m = run[K - 1]
        regs = run

        eq_budget = jnp.zeros((L,), jnp.int32)
        for r in regs:
            eq_budget = eq_budget + jnp.where(r == t, 1, 0)
        eq_seen = jnp.zeros((L,), jnp.int32)
        cnt = obase
        for j in range(N_COLS):
            x = plsc.load_gather(vals_v, [vbase + cint[j]])
            is_eq = x == t
            sel = jnp.logical_or(x > t,
                                 jnp.logical_and(is_eq, eq_seen < eq_budget))
            plsc.store_scatter(idx_v, [cnt], cint[j], mask=sel)
            plsc.store_scatter(val_v, [cnt], x, mask=sel)
            cnt = cnt + jnp.where(sel, 1, 0)
            eq_seen = eq_seen + jnp.where(is_eq, 1, 0)

        es = []
        denom = jnp.zeros((L,), jnp.float32)
        for p in range(K):
            vp = plsc.load_gather(val_v, [obase + cint[p]])
            e = jnp.exp(vp - m)
            es.append(e)
            denom = denom + e
        inv = 1.0 / denom
        for p in range(K):
            plsc.store_scatter(prob_v, [obase + cint[p]], es[p] * inv)

    pltpu.sync_copy(idx_v, idx_hbm.at[pl.ds(base * OPAD, RPW * OPAD)])
    pltpu.sync_copy(prob_v, prob_hbm.at[pl.ds(base * OPAD, RPW * OPAD)])


_sc_call = functools.partial(
    pl.kernel,
    out_type=(
        jax.ShapeDtypeStruct((N_ROWS * OPAD,), jnp.int32),
        jax.ShapeDtypeStruct((N_ROWS * OPAD,), jnp.float32),
    ),
    mesh=plsc.VectorSubcoreMesh(
        core_axis_name="c", subcore_axis_name="s",
        num_cores=NC, num_subcores=NS,
    ),
    compiler_params=pltpu.CompilerParams(needs_layout_passes=False),
    scratch_types=[
        pltpu.VMEM((RPW * VPAD,), jnp.float32),
        pltpu.VMEM((RPW * OPAD,), jnp.int32),
        pltpu.VMEM((RPW * OPAD,), jnp.float32),
        pltpu.VMEM((RPW * OPAD,), jnp.float32),
    ],
)(_sc_body)


def kernel(logits):
    padded = jnp.pad(logits, ((0, 0), (0, VPAD - N_COLS))).reshape(-1)
    idx_p, prob_p = _sc_call(padded)
    return (idx_p.reshape(N_ROWS, OPAD)[:, :K],
            prob_p.reshape(N_ROWS, OPAD)[:, :K])


# speedup vs baseline: 1.1327x; 1.0561x over previous
import functools

import jax
import jax.numpy as jnp
from jax import lax
from jax.experimental import pallas as pl
from jax.experimental.pallas import tpu as pltpu
from jax.experimental.pallas import tpu_sc as plsc

N_ROWS = 32768
N_COLS = 64
K = 8
OPAD = 9
NC = 2
NS = 16
L = 16
NW = NC * NS
RPW = N_ROWS // NW
CHUNK = 512

SORT8_NET = [(0, 1), (2, 3), (4, 5), (6, 7),
             (0, 2), (1, 3), (4, 6), (5, 7),
             (1, 2), (5, 6),
             (0, 4), (1, 5), (2, 6), (3, 7),
             (2, 4), (3, 5),
             (1, 2), (3, 4), (5, 6)]
BITONIC8_NET = [(0, 4), (1, 5), (2, 6), (3, 7),
                (0, 2), (1, 3), (4, 6), (5, 7),
                (0, 1), (2, 3), (4, 5), (6, 7)]


def _apply_net(v, net):
    for i, j in net:
        lo = jnp.minimum(v[i], v[j])
        hi = jnp.maximum(v[i], v[j])
        v[i], v[j] = lo, hi
    return v


def _sc_body(logits_hbm, idx_hbm, prob_hbm, vals_v, padi_v, padv_v,
             cmpi_v, cmpp_v):
    wid = lax.axis_index("s") * NC + lax.axis_index("c")
    base = wid * RPW

    lane = lax.iota(jnp.int32, L)
    pconst = [jnp.full((L,), p, jnp.int32) for p in range(K)]

    def do_chunk(ch):
        pltpu.sync_copy(logits_hbm.at[pl.ds((base + ch * CHUNK) * N_COLS, CHUNK * N_COLS)],
                        vals_v)

        @plsc.parallel_loop(0, CHUNK // L, unroll=1)
        def group(g):
            rows = g * L + lane
            obase = (ch * CHUNK + g * L + lane) * OPAD

            rbase = rows * N_COLS

            def load_step(s):
                cols = (lane + s) & (N_COLS - 1)
                return plsc.load_gather(vals_v, [rbase + cols])

            run = _apply_net([load_step(s) for s in range(K)], SORT8_NET)
            for b in range(1, N_COLS // K):
                blk = _apply_net([load_step(K * b + u) for u in range(K)],
                                 SORT8_NET)
                c = [jnp.maximum(run[i], blk[K - 1 - i]) for i in range(K)]
                run = _apply_net(c, BITONIC8_NET)
            t = run[0]
            m = run[K - 1]

            eq_budget = jnp.zeros((L,), jnp.int32)
            for r in run:
                eq_budget = eq_budget + jnp.where(r == t, 1, 0)
            eq_seen = jnp.zeros((L,), jnp.int32)
            cnt = obase
            for s in range(N_COLS + L - 1):
                cols = s - lane
                if s < L - 1:
                    act = lane <= s
                    cols = jnp.maximum(cols, 0)
                elif s > N_COLS - 1:
                    act = lane >= s - (N_COLS - 1)
                    cols = jnp.minimum(cols, N_COLS - 1)
                else:
                    act = None
                x = plsc.load_gather(vals_v, [rbase + cols])
                is_eq = x == t
                sel = jnp.logical_or(
                    x > t, jnp.logical_and(is_eq, eq_seen < eq_budget))
                if act is not None:
                    is_eq = jnp.logical_and(is_eq, act)
                    sel = jnp.logical_and(sel, act)
                plsc.store_scatter(padi_v, [cnt], cols, mask=sel)
                plsc.store_scatter(padv_v, [cnt], x, mask=sel)
                cnt = cnt + jnp.where(sel, 1, 0)
                eq_seen = eq_seen + jnp.where(is_eq, 1, 0)

            es = []
            denom = jnp.zeros((L,), jnp.float32)
            for p in range(K):
                vp = plsc.load_gather(padv_v, [obase + pconst[p]])
                e = jnp.exp(vp - m)
                es.append(e)
                denom = denom + e
            inv = 1.0 / denom
            for p in range(K):
                plsc.store_scatter(padv_v, [obase + pconst[p]], es[p] * inv)

    for ch in range(RPW // CHUNK):
        do_chunk(ch)

    rvec = lane >> 3
    svec = lane & (K - 1)
    uvec = rvec * OPAD + svec

    @plsc.parallel_loop(0, RPW * K // L, unroll=1)
    def repack(i):
        src = i * (2 * OPAD) + uvec
        rr = 2 * i + rvec
        dst = i * L + lane
        plsc.store_scatter(cmpi_v, [dst],
                           plsc.load_gather(padi_v, [src]))
        plsc.store_scatter(cmpp_v, [dst],
                           plsc.load_gather(padv_v, [src]))

    pltpu.sync_copy(cmpi_v, idx_hbm.at[pl.ds(base * K, RPW * K)])
    pltpu.sync_copy(cmpp_v, prob_hbm.at[pl.ds(base * K, RPW * K)])


_sc_call = functools.partial(
    pl.kernel,
    out_type=(
        jax.ShapeDtypeStruct((N_ROWS * K,), jnp.int32),
        jax.ShapeDtypeStruct((N_ROWS * K,), jnp.float32),
    ),
    mesh=plsc.VectorSubcoreMesh(
        core_axis_name="c", subcore_axis_name="s",
        num_cores=NC, num_subcores=NS,
    ),
    compiler_params=pltpu.CompilerParams(needs_layout_passes=False),
    scratch_types=[
        pltpu.VMEM((CHUNK * N_COLS,), jnp.float32),
        pltpu.VMEM((RPW * OPAD,), jnp.int32),
        pltpu.VMEM((RPW * OPAD,), jnp.float32),
        pltpu.VMEM((RPW * K,), jnp.int32),
        pltpu.VMEM((RPW * K,), jnp.float32),
    ],
)(_sc_body)


def kernel(logits):
    idx_f, prob_f = _sc_call(logits.reshape(-1))
    return idx_f.reshape(N_ROWS, K), prob_f.reshape(N_ROWS, K)


# speedup vs baseline: 1.2513x; 1.1047x over previous
import functools

import jax
import jax.numpy as jnp
from jax import lax
from jax.experimental import pallas as pl
from jax.experimental.pallas import tpu as pltpu
from jax.experimental.pallas import tpu_sc as plsc

N_ROWS = 32768
N_COLS = 64
K = 8
VPAD = 65
OPAD = 9
NC = 2
NS = 16
L = 16
NW = NC * NS
RPW = N_ROWS // NW
CHUNK = 256
NCH = RPW // CHUNK

SORT8_NET = [(0, 1), (2, 3), (4, 5), (6, 7),
             (0, 2), (1, 3), (4, 6), (5, 7),
             (1, 2), (5, 6),
             (0, 4), (1, 5), (2, 6), (3, 7),
             (2, 4), (3, 5),
             (1, 2), (3, 4), (5, 6)]
BITONIC8_NET = [(0, 4), (1, 5), (2, 6), (3, 7),
                (0, 2), (1, 3), (4, 6), (5, 7),
                (0, 1), (2, 3), (4, 5), (6, 7)]


def _apply_net(v, net):
    for i, j in net:
        lo = jnp.minimum(v[i], v[j])
        hi = jnp.maximum(v[i], v[j])
        v[i], v[j] = lo, hi
    return v


def _sc_body(logits_hbm, idx_hbm, prob_hbm, raw0_v, raw1_v, vals_v,
             padi_v, padv_v, cmpi_v, cmpp_v, sem0, sem1):
    wid = lax.axis_index("s") * NC + lax.axis_index("c")
    base = wid * RPW

    lane = lax.iota(jnp.int32, L)
    cint = [jnp.full((L,), j, jnp.int32) for j in range(N_COLS)]

    raws = [raw0_v, raw1_v]
    sems = [sem0, sem1]

    def start_fetch(ch):
        src = logits_hbm.at[pl.ds((base + ch * CHUNK) * N_COLS,
                                  CHUNK * N_COLS)]
        return pltpu.async_copy(src, raws[ch % 2], sems[ch % 2])

    pending = start_fetch(0)
    for ch in range(NCH):
        nxt = start_fetch(ch + 1) if ch + 1 < NCH else None
        pending.wait()
        raw_v = raws[ch % 2]
        pending = nxt

        @plsc.parallel_loop(0, CHUNK, unroll=1)
        def restride(r):
            for q in range(N_COLS // L):
                vals_v[pl.ds(r * VPAD + q * L, L)] = \
                    raw_v[pl.ds(r * N_COLS + q * L, L)]

        @plsc.parallel_loop(0, CHUNK // L, unroll=1)
        def group(g):
            vbase = (g * L + lane) * VPAD
            obase = (ch * CHUNK + g * L + lane) * OPAD

            def load_col(j):
                return plsc.load_gather(vals_v, [vbase + cint[j]])

            run = _apply_net([load_col(u) for u in range(K)], SORT8_NET)
            for b in range(1, N_COLS // K):
                blk = _apply_net([load_col(K * b + u) for u in range(K)],
                                 SORT8_NET)
                c = [jnp.maximum(run[i], blk[K - 1 - i]) for i in range(K)]
                run = _apply_net(c, BITONIC8_NET)
            t = run[0]
            m = run[K - 1]

            eq_budget = jnp.zeros((L,), jnp.int32)
            for r in run:
                eq_budget = eq_budget + jnp.where(r == t, 1, 0)
            eq_seen = jnp.zeros((L,), jnp.int32)
            cnt = obase
            for j in range(N_COLS):
                x = load_col(j)
                is_eq = x == t
                sel = jnp.logical_or(
                    x > t, jnp.logical_and(is_eq, eq_seen < eq_budget))
                plsc.store_scatter(padi_v, [cnt], cint[j], mask=sel)
                plsc.store_scatter(padv_v, [cnt], x, mask=sel)
                cnt = cnt + jnp.where(sel, 1, 0)
                eq_seen = eq_seen + jnp.where(is_eq, 1, 0)

            es = []
            denom = jnp.zeros((L,), jnp.float32)
            for p in range(K):
                vp = plsc.load_gather(padv_v, [obase + cint[p]])
                e = jnp.exp(vp - m)
                es.append(e)
                denom = denom + e
            inv = 1.0 / denom
            for p in range(K):
                plsc.store_scatter(padv_v, [obase + cint[p]], es[p] * inv)

    rvec = lane >> 3
    svec = lane & (K - 1)
    uvec = rvec * OPAD + svec

    @plsc.parallel_loop(0, RPW * K // L, unroll=1)
    def repack(i):
        src = i * (2 * OPAD) + uvec
        dst = i * L + lane
        plsc.store_scatter(cmpi_v, [dst],
                           plsc.load_gather(padi_v, [src]))
        plsc.store_scatter(cmpp_v, [dst],
                           plsc.load_gather(padv_v, [src]))

    pltpu.sync_copy(cmpi_v, idx_hbm.at[pl.ds(base * K, RPW * K)])
    pltpu.sync_copy(cmpp_v, prob_hbm.at[pl.ds(base * K, RPW * K)])


_sc_call = functools.partial(
    pl.kernel,
    out_type=(
        jax.ShapeDtypeStruct((N_ROWS * K,), jnp.int32),
        jax.ShapeDtypeStruct((N_ROWS * K,), jnp.float32),
    ),
    mesh=plsc.VectorSubcoreMesh(
        core_axis_name="c", subcore_axis_name="s",
        num_cores=NC, num_subcores=NS,
    ),
    compiler_params=pltpu.CompilerParams(needs_layout_passes=False),
    scratch_types=[
        pltpu.VMEM((CHUNK * N_COLS,), jnp.float32),
        pltpu.VMEM((CHUNK * N_COLS,), jnp.float32),
        pltpu.VMEM((CHUNK * VPAD,), jnp.float32),
        pltpu.VMEM((RPW * OPAD,), jnp.int32),
        pltpu.VMEM((RPW * OPAD,), jnp.float32),
        pltpu.VMEM((RPW * K,), jnp.int32),
        pltpu.VMEM((RPW * K,), jnp.float32),
        pltpu.SemaphoreType.DMA,
        pltpu.SemaphoreType.DMA,
    ],
)(_sc_body)


def kernel(logits):
    idx_f, prob_f = _sc_call(logits.reshape(-1))
    return idx_f.reshape(N_ROWS, K), prob_f.reshape(N_ROWS, K)


# speedup vs baseline: 1.2568x; 1.0044x over previous
import functools

import jax
import jax.numpy as jnp
from jax import lax
from jax.experimental import pallas as pl
from jax.experimental.pallas import tpu as pltpu
from jax.experimental.pallas import tpu_sc as plsc

N_ROWS = 32768
N_COLS = 64
K = 8
VPAD = 65
OPAD = 9
NC = 2
NS = 16
L = 16
NW = NC * NS
RPW = N_ROWS // NW
CHUNK = 256
NCH = RPW // CHUNK

SORT8_NET = [(0, 1), (2, 3), (4, 5), (6, 7),
             (0, 2), (1, 3), (4, 6), (5, 7),
             (1, 2), (5, 6),
             (0, 4), (1, 5), (2, 6), (3, 7),
             (2, 4), (3, 5),
             (1, 2), (3, 4), (5, 6)]
BITONIC8_NET = [(0, 4), (1, 5), (2, 6), (3, 7),
                (0, 2), (1, 3), (4, 6), (5, 7),
                (0, 1), (2, 3), (4, 5), (6, 7)]


def _apply_net(v, net):
    for i, j in net:
        lo = jnp.minimum(v[i], v[j])
        hi = jnp.maximum(v[i], v[j])
        v[i], v[j] = lo, hi
    return v


def _sc_body(logits_hbm, idx_hbm, prob_hbm, raw0_v, raw1_v, vals_v,
             padi_v, padv_v, cmpi_v, cmpp_v, sem0, sem1):
    wid = lax.axis_index("s") * NC + lax.axis_index("c")
    base = wid * RPW

    lane = lax.iota(jnp.int32, L)
    cint = [jnp.full((L,), j, jnp.int32) for j in range(N_COLS)]

    raws = [raw0_v, raw1_v]
    sems = [sem0, sem1]

    def start_fetch(ch):
        src = logits_hbm.at[pl.ds((base + ch * CHUNK) * N_COLS,
                                  CHUNK * N_COLS)]
        return pltpu.async_copy(src, raws[ch % 2], sems[ch % 2])

    pending = start_fetch(0)
    for ch in range(NCH):
        nxt = start_fetch(ch + 1) if ch + 1 < NCH else None
        pending.wait()
        raw_v = raws[ch % 2]
        pending = nxt

        @plsc.parallel_loop(0, CHUNK, unroll=1)
        def restride(r):
            for q in range(N_COLS // L):
                vals_v[pl.ds(r * VPAD + q * L, L)] = \
                    raw_v[pl.ds(r * N_COLS + q * L, L)]

        @plsc.parallel_loop(0, CHUNK // L, unroll=1)
        def group(g):
            vbase = (g * L + lane) * VPAD
            obase = (ch * CHUNK + g * L + lane) * OPAD

            def load_col(j):
                return plsc.load_gather(vals_v, [vbase + cint[j]])

            run = _apply_net([load_col(u) for u in range(K)], SORT8_NET)
            for b in range(1, N_COLS // K):
                blk = _apply_net([load_col(K * b + u) for u in range(K)],
                                 SORT8_NET)
                c = [jnp.maximum(run[i], blk[K - 1 - i]) for i in range(K)]
                run = _apply_net(c, BITONIC8_NET)
            t = run[0]
            m = run[K - 1]

            eq_budget = jnp.zeros((L,), jnp.int32)
            for r in run:
                eq_budget = eq_budget + jnp.where(r == t, 1, 0)
            eq_seen = jnp.zeros((L,), jnp.int32)
            cnt = obase
            for j in range(N_COLS):
                x = load_col(j)
                is_eq = x == t
                sel = jnp.logical_or(
                    x > t, jnp.logical_and(is_eq, eq_seen < eq_budget))
                plsc.store_scatter(padi_v, [cnt], cint[j], mask=sel)
                plsc.store_scatter(padv_v, [cnt], x, mask=sel)
                cnt = cnt + jnp.where(sel, 1, 0)
                eq_seen = eq_seen + jnp.where(is_eq, 1, 0)

            es = []
            denom = jnp.zeros((L,), jnp.float32)
            for p in range(K):
                vp = plsc.load_gather(padv_v, [obase + cint[p]])
                e = jnp.exp(vp - m)
                es.append(e)
                denom = denom + e
            inv = 1.0 / denom
            for p in range(K):
                plsc.store_scatter(padv_v, [obase + cint[p]], es[p] * inv)

    rvec = lane >> 3
    svec = lane & (K - 1)
    uvec = rvec * OPAD + svec

    @plsc.parallel_loop(0, RPW * K // L, unroll=1)
    def repack(i):
        src = i * (2 * OPAD) + uvec
        dst_r = (i * L) >> 7
        dst_c = (i * L) & 127
        plsc.store_scatter(cmpi_v, [jnp.full((L,), 0, jnp.int32) + dst_r,
                                    dst_c + lane],
                           plsc.load_gather(padi_v, [src]))
        plsc.store_scatter(cmpp_v, [jnp.full((L,), 0, jnp.int32) + dst_r,
                                    dst_c + lane],
                           plsc.load_gather(padv_v, [src]))

    obeg = pl.multiple_of(base * K // 128, 8)
    pltpu.sync_copy(cmpi_v, idx_hbm.at[pl.ds(obeg, RPW * K // 128)])
    pltpu.sync_copy(cmpp_v, prob_hbm.at[pl.ds(obeg, RPW * K // 128)])


_sc_call = functools.partial(
    pl.kernel,
    out_type=(
        jax.ShapeDtypeStruct((N_ROWS * K // 128, 128), jnp.int32),
        jax.ShapeDtypeStruct((N_ROWS * K // 128, 128), jnp.float32),
    ),
    mesh=plsc.VectorSubcoreMesh(
        core_axis_name="c", subcore_axis_name="s",
        num_cores=NC, num_subcores=NS,
    ),
    compiler_params=pltpu.CompilerParams(needs_layout_passes=False),
    scratch_types=[
        pltpu.VMEM((CHUNK * N_COLS,), jnp.float32),
        pltpu.VMEM((CHUNK * N_COLS,), jnp.float32),
        pltpu.VMEM((CHUNK * VPAD,), jnp.float32),
        pltpu.VMEM((RPW * OPAD,), jnp.int32),
        pltpu.VMEM((RPW * OPAD,), jnp.float32),
        pltpu.VMEM((RPW * K // 128, 128), jnp.int32),
        pltpu.VMEM((RPW * K // 128, 128), jnp.float32),
        pltpu.SemaphoreType.DMA,
        pltpu.SemaphoreType.DMA,
    ],
)(_sc_body)


def kernel(logits):
    idx_f, prob_f = _sc_call(logits.reshape(-1))
    return idx_f.reshape(N_ROWS, K), prob_f.reshape(N_ROWS, K)


# speedup vs baseline: 1.2581x; 1.0011x over previous
import functools

import jax
import jax.numpy as jnp
from jax import lax
from jax.experimental import pallas as pl
from jax.experimental.pallas import tpu as pltpu
from jax.experimental.pallas import tpu_sc as plsc

N_ROWS = 32768
N_COLS = 64
K = 8
VPAD = 65
OPAD = 9
NC = 2
NS = 16
L = 16
NW = NC * NS
RPW = N_ROWS // NW
CHUNK = 256
NCH = RPW // CHUNK

SORT8_NET = [(0, 1), (2, 3), (4, 5), (6, 7),
             (0, 2), (1, 3), (4, 6), (5, 7),
             (1, 2), (5, 6),
             (0, 4), (1, 5), (2, 6), (3, 7),
             (2, 4), (3, 5),
             (1, 2), (3, 4), (5, 6)]
BITONIC8_NET = [(0, 4), (1, 5), (2, 6), (3, 7),
                (0, 2), (1, 3), (4, 6), (5, 7),
                (0, 1), (2, 3), (4, 5), (6, 7)]


def _apply_net(v, net):
    for i, j in net:
        lo = jnp.minimum(v[i], v[j])
        hi = jnp.maximum(v[i], v[j])
        v[i], v[j] = lo, hi
    return v


def _sc_body(logits_hbm, idx_hbm, prob_hbm, raw0_v, raw1_v, vals_v,
             padi_v, padv_v, cmpi_v, cmpp_v, sem0, sem1):
    wid = lax.axis_index("s") * NC + lax.axis_index("c")
    base = wid * RPW

    lane = lax.iota(jnp.int32, L)
    cint = [jnp.full((L,), j, jnp.int32) for j in range(N_COLS)]

    raws = [raw0_v, raw1_v]
    sems = [sem0, sem1]

    def start_fetch(ch):
        r0 = pl.multiple_of((base + ch * CHUNK) * N_COLS // 128, 8)
        src = logits_hbm.at[pl.ds(r0, CHUNK * N_COLS // 128)]
        return pltpu.async_copy(src, raws[ch % 2], sems[ch % 2])

    pending = start_fetch(0)
    for ch in range(NCH):
        nxt = start_fetch(ch + 1) if ch + 1 < NCH else None
        pending.wait()
        raw_v = raws[ch % 2]
        pending = nxt

        @plsc.parallel_loop(0, CHUNK, unroll=1)
        def restride(r):
            for q in range(N_COLS // L):
                w = r * N_COLS + q * L
                vals_v[pl.ds(r * VPAD + q * L, L)] = \
                    raw_v[w >> 7, pl.ds((q * L) % 128, L)] if False else \
                    raw_v[w >> 7, pl.ds(((r & 1) * N_COLS + q * L), L)]

        @plsc.parallel_loop(0, CHUNK // L, unroll=1)
        def group(g):
            vbase = (g * L + lane) * VPAD
            obase = (ch * CHUNK + g * L + lane) * OPAD

            def load_col(j):
                return plsc.load_gather(vals_v, [vbase + cint[j]])

            run = _apply_net([load_col(u) for u in range(K)], SORT8_NET)
            for b in range(1, N_COLS // K):
                blk = _apply_net([load_col(K * b + u) for u in range(K)],
                                 SORT8_NET)
                c = [jnp.maximum(run[i], blk[K - 1 - i]) for i in range(K)]
                run = _apply_net(c, BITONIC8_NET)
            t = run[0]
            m = run[K - 1]

            eq_budget = jnp.zeros((L,), jnp.int32)
            for r in run:
                eq_budget = eq_budget + jnp.where(r == t, 1, 0)
            eq_seen = jnp.zeros((L,), jnp.int32)
            cnt = obase
            for j in range(N_COLS):
                x = load_col(j)
                is_eq = x == t
                sel = jnp.logical_or(
                    x > t, jnp.logical_and(is_eq, eq_seen < eq_budget))
                plsc.store_scatter(padi_v, [cnt], cint[j], mask=sel)
                plsc.store_scatter(padv_v, [cnt], x, mask=sel)
                cnt = cnt + jnp.where(sel, 1, 0)
                eq_seen = eq_seen + jnp.where(is_eq, 1, 0)

            es = []
            denom = jnp.zeros((L,), jnp.float32)
            for p in range(K):
                vp = plsc.load_gather(padv_v, [obase + cint[p]])
                e = jnp.exp(vp - m)
                es.append(e)
                denom = denom + e
            inv = 1.0 / denom
            for p in range(K):
                plsc.store_scatter(padv_v, [obase + cint[p]], es[p] * inv)

    rvec = lane >> 3
    svec = lane & (K - 1)
    uvec = rvec * OPAD + svec

    @plsc.parallel_loop(0, RPW * K // L, unroll=1)
    def repack(i):
        src = i * (2 * OPAD) + uvec
        dst_r = (i * L) >> 7
        dst_c = (i * L) & 127
        plsc.store_scatter(cmpi_v, [jnp.full((L,), 0, jnp.int32) + dst_r,
                                    dst_c + lane],
                           plsc.load_gather(padi_v, [src]))
        plsc.store_scatter(cmpp_v, [jnp.full((L,), 0, jnp.int32) + dst_r,
                                    dst_c + lane],
                           plsc.load_gather(padv_v, [src]))

    obeg = pl.multiple_of(base * K // 128, 8)
    pltpu.sync_copy(cmpi_v, idx_hbm.at[pl.ds(obeg, RPW * K // 128)])
    pltpu.sync_copy(cmpp_v, prob_hbm.at[pl.ds(obeg, RPW * K // 128)])


_sc_call = functools.partial(
    pl.kernel,
    out_type=(
        jax.ShapeDtypeStruct((N_ROWS * K // 128, 128), jnp.int32),
        jax.ShapeDtypeStruct((N_ROWS * K // 128, 128), jnp.float32),
    ),
    mesh=plsc.VectorSubcoreMesh(
        core_axis_name="c", subcore_axis_name="s",
        num_cores=NC, num_subcores=NS,
    ),
    compiler_params=pltpu.CompilerParams(needs_layout_passes=False),
    scratch_types=[
        pltpu.VMEM((CHUNK * N_COLS // 128, 128), jnp.float32),
        pltpu.VMEM((CHUNK * N_COLS // 128, 128), jnp.float32),
        pltpu.VMEM((CHUNK * VPAD,), jnp.float32),
        pltpu.VMEM((RPW * OPAD,), jnp.int32),
        pltpu.VMEM((RPW * OPAD,), jnp.float32),
        pltpu.VMEM((RPW * K // 128, 128), jnp.int32),
        pltpu.VMEM((RPW * K // 128, 128), jnp.float32),
        pltpu.SemaphoreType.DMA,
        pltpu.SemaphoreType.DMA,
    ],
)(_sc_body)


def kernel(logits):
    idx_f, prob_f = _sc_call(logits.reshape(N_ROWS * N_COLS // 128, 128))
    return idx_f.reshape(N_ROWS, K), prob_f.reshape(N_ROWS, K)


# speedup vs baseline: 1.2685x; 1.0083x over previous
import functools

import jax
import jax.numpy as jnp
from jax import lax
from jax.experimental import pallas as pl
from jax.experimental.pallas import tpu as pltpu
from jax.experimental.pallas import tpu_sc as plsc

N_ROWS = 32768
N_COLS = 64
K = 8
VPAD = 65
OPAD = 9
NC = 2
NS = 16
L = 16
NW = NC * NS
RPW = N_ROWS // NW
CHUNK = 512
NCH = RPW // CHUNK

SORT8_NET = [(0, 1), (2, 3), (4, 5), (6, 7),
             (0, 2), (1, 3), (4, 6), (5, 7),
             (1, 2), (5, 6),
             (0, 4), (1, 5), (2, 6), (3, 7),
             (2, 4), (3, 5),
             (1, 2), (3, 4), (5, 6)]
BITONIC8_NET = [(0, 4), (1, 5), (2, 6), (3, 7),
                (0, 2), (1, 3), (4, 6), (5, 7),
                (0, 1), (2, 3), (4, 5), (6, 7)]


def _apply_net(v, net):
    for i, j in net:
        lo = jnp.minimum(v[i], v[j])
        hi = jnp.maximum(v[i], v[j])
        v[i], v[j] = lo, hi
    return v


def _sc_body(logits_hbm, idx_hbm, prob_hbm, raw0_v, vals_v,
             padi_v, padv_v, cmpi_v, cmpp_v):
    wid = lax.axis_index("s") * NC + lax.axis_index("c")
    base = wid * RPW

    lane = lax.iota(jnp.int32, L)
    cint = [jnp.full((L,), j, jnp.int32) for j in range(N_COLS)]

    def fetch(ch):
        r0 = pl.multiple_of((base + ch * CHUNK) * N_COLS // 128, 8)
        src = logits_hbm.at[pl.ds(r0, CHUNK * N_COLS // 128)]
        pltpu.sync_copy(src, raw0_v)

    for ch in range(NCH):
        fetch(ch)
        raw_v = raw0_v

        @plsc.parallel_loop(0, CHUNK, unroll=1)
        def restride(r):
            for q in range(N_COLS // L):
                w = r * N_COLS + q * L
                vals_v[pl.ds(r * VPAD + q * L, L)] = \
                    raw_v[w >> 7, pl.ds((q * L) % 128, L)] if False else \
                    raw_v[w >> 7, pl.ds(((r & 1) * N_COLS + q * L), L)]

        @plsc.parallel_loop(0, CHUNK // L, unroll=1)
        def group(g):
            vbase = (g * L + lane) * VPAD
            obase = (ch * CHUNK + g * L + lane) * OPAD

            def load_col(j):
                return plsc.load_gather(vals_v, [vbase + cint[j]])

            run = _apply_net([load_col(u) for u in range(K)], SORT8_NET)
            for b in range(1, N_COLS // K):
                blk = _apply_net([load_col(K * b + u) for u in range(K)],
                                 SORT8_NET)
                c = [jnp.maximum(run[i], blk[K - 1 - i]) for i in range(K)]
                run = _apply_net(c, BITONIC8_NET)
            t = run[0]
            m = run[K - 1]

            eq_budget = jnp.zeros((L,), jnp.int32)
            for r in run:
                eq_budget = eq_budget + jnp.where(r == t, 1, 0)
            eq_seen = jnp.zeros((L,), jnp.int32)
            cnt = obase
            for j in range(N_COLS):
                x = load_col(j)
                is_eq = x == t
                sel = jnp.logical_or(
                    x > t, jnp.logical_and(is_eq, eq_seen < eq_budget))
                plsc.store_scatter(padi_v, [cnt], cint[j], mask=sel)
                plsc.store_scatter(padv_v, [cnt], x, mask=sel)
                cnt = cnt + jnp.where(sel, 1, 0)
                eq_seen = eq_seen + jnp.where(is_eq, 1, 0)

            es = []
            denom = jnp.zeros((L,), jnp.float32)
            for p in range(K):
                vp = plsc.load_gather(padv_v, [obase + cint[p]])
                e = jnp.exp(vp - m)
                es.append(e)
                denom = denom + e
            inv = 1.0 / denom
            for p in range(K):
                plsc.store_scatter(padv_v, [obase + cint[p]], es[p] * inv)

    rvec = lane >> 3
    svec = lane & (K - 1)
    uvec = rvec * OPAD + svec

    @plsc.parallel_loop(0, RPW * K // L, unroll=1)
    def repack(i):
        src = i * (2 * OPAD) + uvec
        dst_r = (i * L) >> 7
        dst_c = (i * L) & 127
        plsc.store_scatter(cmpi_v, [jnp.full((L,), 0, jnp.int32) + dst_r,
                                    dst_c + lane],
                           plsc.load_gather(padi_v, [src]))
        plsc.store_scatter(cmpp_v, [jnp.full((L,), 0, jnp.int32) + dst_r,
                                    dst_c + lane],
                           plsc.load_gather(padv_v, [src]))

    obeg = pl.multiple_of(base * K // 128, 8)
    pltpu.sync_copy(cmpi_v, idx_hbm.at[pl.ds(obeg, RPW * K // 128)])
    pltpu.sync_copy(cmpp_v, prob_hbm.at[pl.ds(obeg, RPW * K // 128)])


_sc_call = functools.partial(
    pl.kernel,
    out_type=(
        jax.ShapeDtypeStruct((N_ROWS * K // 128, 128), jnp.int32),
        jax.ShapeDtypeStruct((N_ROWS * K // 128, 128), jnp.float32),
    ),
    mesh=plsc.VectorSubcoreMesh(
        core_axis_name="c", subcore_axis_name="s",
        num_cores=NC, num_subcores=NS,
    ),
    compiler_params=pltpu.CompilerParams(needs_layout_passes=False),
    scratch_types=[
        pltpu.VMEM((CHUNK * N_COLS // 128, 128), jnp.float32),
        pltpu.VMEM((CHUNK * VPAD,), jnp.float32),
        pltpu.VMEM((RPW * OPAD,), jnp.int32),
        pltpu.VMEM((RPW * OPAD,), jnp.float32),
        pltpu.VMEM((RPW * K // 128, 128), jnp.int32),
        pltpu.VMEM((RPW * K // 128, 128), jnp.float32),
    ],
)(_sc_body)


def kernel(logits):
    idx_f, prob_f = _sc_call(logits.reshape(N_ROWS * N_COLS // 128, 128))
    return idx_f.reshape(N_ROWS, K), prob_f.reshape(N_ROWS, K)


# speedup vs baseline: 1.2709x; 1.0019x over previous
import functools

import jax
import jax.numpy as jnp
from jax import lax
from jax.experimental import pallas as pl
from jax.experimental.pallas import tpu as pltpu
from jax.experimental.pallas import tpu_sc as plsc

N_ROWS = 32768
N_COLS = 64
K = 8
VPAD = 65
OPAD = 9
NC = 2
NS = 16
L = 16
NW = NC * NS
RPW = N_ROWS // NW
CHUNK = 512
NCH = RPW // CHUNK

SORT8_NET = [(0, 1), (2, 3), (4, 5), (6, 7),
             (0, 2), (1, 3), (4, 6), (5, 7),
             (1, 2), (5, 6),
             (0, 4), (1, 5), (2, 6), (3, 7),
             (2, 4), (3, 5),
             (1, 2), (3, 4), (5, 6)]
BITONIC8_NET = [(0, 4), (1, 5), (2, 6), (3, 7),
                (0, 2), (1, 3), (4, 6), (5, 7),
                (0, 1), (2, 3), (4, 5), (6, 7)]


def _apply_net(v, net):
    for i, j in net:
        lo = jnp.minimum(v[i], v[j])
        hi = jnp.maximum(v[i], v[j])
        v[i], v[j] = lo, hi
    return v


def _sc_body(logits_hbm, idx_hbm, prob_hbm, raw0_v, vals_v,
             padi_v, padv_v, cmpi_v, cmpp_v):
    wid = lax.axis_index("s") * NC + lax.axis_index("c")
    base = wid * RPW

    lane = lax.iota(jnp.int32, L)
    cint = [jnp.full((L,), j, jnp.int32) for j in range(N_COLS)]

    def fetch(ch):
        r0 = pl.multiple_of((base + ch * CHUNK) * N_COLS // 128, 8)
        src = logits_hbm.at[pl.ds(r0, CHUNK * N_COLS // 128)]
        pltpu.sync_copy(src, raw0_v)

    for ch in range(NCH):
        fetch(ch)
        raw_v = raw0_v

        @plsc.parallel_loop(0, CHUNK, unroll=1)
        def restride(r):
            for q in range(N_COLS // L):
                w = r * N_COLS + q * L
                vals_v[pl.ds(r * VPAD + q * L, L)] = \
                    raw_v[w >> 7, pl.ds((r & 1) * N_COLS + q * L, L)]

        @plsc.parallel_loop(0, CHUNK // L, unroll=1)
        def group(g):
            vbase = (g * L + lane) * VPAD
            obase = (ch * CHUNK + g * L + lane) * OPAD

            def load_col(j):
                return plsc.load_gather(vals_v, [vbase + cint[j]])

            run = _apply_net([load_col(u) for u in range(K)], SORT8_NET)
            for b in range(1, N_COLS // K):
                blk = _apply_net([load_col(K * b + u) for u in range(K)],
                                 SORT8_NET)
                c = [jnp.maximum(run[i], blk[K - 1 - i]) for i in range(K)]
                run = _apply_net(c, BITONIC8_NET)
            t = run[0]
            m = run[K - 1]

            eq_budget = jnp.zeros((L,), jnp.int32)
            for r in run:
                eq_budget = eq_budget + jnp.where(r == t, 1, 0)
            eq_seen = jnp.zeros((L,), jnp.int32)
            cnt = obase
            for j in range(N_COLS):
                x = load_col(j)
                is_eq = x == t
                sel = jnp.logical_or(
                    x > t, jnp.logical_and(is_eq, eq_seen < eq_budget))
                plsc.store_scatter(padi_v, [cnt], cint[j], mask=sel)
                plsc.store_scatter(padv_v, [cnt], x, mask=sel)
                cnt = cnt + jnp.where(sel, 1, 0)
                eq_seen = eq_seen + jnp.where(is_eq, 1, 0)

            es = []
            denom = jnp.zeros((L,), jnp.float32)
            for p in range(K):
                vp = plsc.load_gather(padv_v, [obase + cint[p]])
                e = jnp.exp(vp - m)
                es.append(e)
                denom = denom + e
            inv = 1.0 / denom
            for p in range(K):
                plsc.store_scatter(padv_v, [obase + cint[p]], es[p] * inv)

    rvec = lane >> 3
    svec = lane & (K - 1)
    uvec = rvec * OPAD + svec

    @plsc.parallel_loop(0, RPW * K // L, unroll=1)
    def repack(i):
        src = i * (2 * OPAD) + uvec
        dst_r = (i * L) >> 7
        dst_c = (i * L) & 127
        plsc.store_scatter(cmpi_v, [jnp.full((L,), 0, jnp.int32) + dst_r,
                                    dst_c + lane],
                           plsc.load_gather(padi_v, [src]))
        plsc.store_scatter(cmpp_v, [jnp.full((L,), 0, jnp.int32) + dst_r,
                                    dst_c + lane],
                           plsc.load_gather(padv_v, [src]))

    obeg = pl.multiple_of(base * K // 128, 8)
    pltpu.sync_copy(cmpi_v, idx_hbm.at[pl.ds(obeg, RPW * K // 128)])
    pltpu.sync_copy(cmpp_v, prob_hbm.at[pl.ds(obeg, RPW * K // 128)])


_sc_call = functools.partial(
    pl.kernel,
    out_type=(
        jax.ShapeDtypeStruct((N_ROWS * K // 128, 128), jnp.int32),
        jax.ShapeDtypeStruct((N_ROWS * K // 128, 128), jnp.float32),
    ),
    mesh=plsc.VectorSubcoreMesh(
        core_axis_name="c", subcore_axis_name="s",
        num_cores=NC, num_subcores=NS,
    ),
    compiler_params=pltpu.CompilerParams(needs_layout_passes=False),
    scratch_types=[
        pltpu.VMEM((CHUNK * N_COLS // 128, 128), jnp.float32),
        pltpu.VMEM((CHUNK * VPAD,), jnp.float32),
        pltpu.VMEM((RPW * OPAD,), jnp.int32),
        pltpu.VMEM((RPW * OPAD,), jnp.float32),
        pltpu.VMEM((RPW * K // 128, 128), jnp.int32),
        pltpu.VMEM((RPW * K // 128, 128), jnp.float32),
    ],
)(_sc_body)


def kernel(logits):
    idx_f, prob_f = _sc_call(logits.reshape(N_ROWS * N_COLS // 128, 128))
    return idx_f.reshape(N_ROWS, K), prob_f.reshape(N_ROWS, K)
